# Initial kernel scaffold; baseline (speedup 1.0000x reference)
import jax
import jax.numpy as jnp
from jax.experimental import pallas as pl


def kernel(x, edge_index, edge_weight, Wk, bk, Wq, bq, Wv, bv, Wa, ba, rel_att, rel_msg, rel_pri, skip, ln_g, ln_b):
    raise NotImplementedError("write your pallas kernel here")


# speedup vs baseline: 12.8049x; 12.8049x over previous
import functools

import jax
import jax.numpy as jnp
import numpy as np
from jax import lax
from jax.experimental import pallas as pl
from jax.experimental.pallas import tpu as pltpu
from jax.experimental.pallas import tpu_sc as plsc

N = 10000
E = 320000
D = 128
H = 8
DK = D // H
EPS = 1e-9
LN_EPS = 1e-5

NC = 2
NS = 16
NW = NC * NS
NPAD = 10240
ROWS = NPAD // NS
ET = E // NW
CA = 400
CH = 80
NCH = ET // CH

_I32 = jnp.int32
_F32 = jnp.float32


def _proj_body(x_ref, wq_ref, bq_ref, wk_ref, bk_ref, wv_ref, bv_ref,
               ra_ref, rm_ref, q_ref, k_ref, v_ref):
    x = x_ref[...]
    q_ref[...] = jnp.dot(x, wq_ref[...], preferred_element_type=_F32) + bq_ref[...]
    xk = jnp.dot(x, wk_ref[...], preferred_element_type=_F32) + bk_ref[...]
    xv = jnp.dot(x, wv_ref[...], preferred_element_type=_F32) + bv_ref[...]
    k_ref[...] = jnp.concatenate(
        [jnp.dot(xk[:, h * DK:(h + 1) * DK], ra_ref[h],
                 preferred_element_type=_F32) for h in range(H)], axis=1)
    v_ref[...] = jnp.concatenate(
        [jnp.dot(xv[:, h * DK:(h + 1) * DK], rm_ref[h],
                 preferred_element_type=_F32) for h in range(H)], axis=1)


_BN = 2000


def _proj(x, Wq, bq, Wk, bk, Wv, bv, rel_att, rel_msg):
    sds = jax.ShapeDtypeStruct((N, D), _F32)
    row_spec = pl.BlockSpec((_BN, D), lambda i: (i, 0))
    full = pl.BlockSpec((D, D), lambda i: (0, 0))
    bias = pl.BlockSpec((1, D), lambda i: (0, 0))
    rel = pl.BlockSpec((H, DK, DK), lambda i: (0, 0, 0))
    return pl.pallas_call(
        _proj_body,
        grid=(N // _BN,),
        in_specs=[row_spec, full, bias, full, bias, full, bias, rel, rel],
        out_specs=(row_spec, row_spec, row_spec),
        out_shape=(sds, sds, sds),
    )(x, Wq, bq.reshape(1, D), Wk, bk.reshape(1, D), Wv, bv.reshape(1, D),
      rel_att, rel_msg)


def _ln16(xv):
    b = plsc.bitcast(xv, _I32)
    ex = (b >> jnp.full((16,), 23, _I32)) - jnp.full((16,), 127, _I32)
    mb = (b & jnp.full((16,), 0x007FFFFF, _I32)) | jnp.full((16,), 0x3F800000, _I32)
    m = plsc.bitcast(mb, _F32)
    adj = m > jnp.full((16,), 1.4142135, _F32)
    m = jnp.where(adj, m * jnp.full((16,), 0.5, _F32), m)
    ex = jnp.where(adj, ex + jnp.full((16,), 1, _I32), ex)
    one = jnp.full((16,), 1.0, _F32)
    z = (m - one) / (m + one)
    z2 = z * z
    lnm = jnp.full((16,), 2.0, _F32) * z * (
        one + z2 * (jnp.full((16,), 1.0 / 3.0, _F32)
                    + z2 * (jnp.full((16,), 0.2, _F32)
                            + z2 * jnp.full((16,), 1.0 / 7.0, _F32))))
    return ex.astype(_F32) * jnp.full((16,), 0.6931471805599453, _F32) + lnm


_MESH = plsc.VectorSubcoreMesh(core_axis_name="c", subcore_axis_name="s")


@functools.partial(
    pl.kernel,
    out_type=jax.ShapeDtypeStruct((NC, NPAD), _F32),
    mesh=_MESH,
    compiler_params=pltpu.CompilerParams(needs_layout_passes=False),
    scratch_types=[
        pltpu.VMEM((CA,), _I32),
        pltpu.VMEM((CA,), _F32),
        pltpu.VMEM((ROWS,), _F32),
        pltpu.VMEM_SHARED((NPAD,), _F32),
    ],
)
def _phase_a(dst_hbm, ew_hbm, den_hbm, dstb, ewb, outb, den_sh):
    c = lax.axis_index("c")
    s = lax.axis_index("s")
    wid = s * NC + c
    zeros16f = jnp.zeros((16,), _F32)

    def zloop(i, _):
        outb[pl.ds(pl.multiple_of(i * 16, 16), 16)] = zeros16f
        return 0
    lax.fori_loop(0, ROWS // 16, zloop, 0)
    pltpu.sync_copy(outb, den_sh.at[pl.ds(s * ROWS, ROWS)])
    plsc.subcore_barrier()

    base = wid * ET

    def chunk(ci, _):
        off = base + ci * CA
        pltpu.sync_copy(dst_hbm.at[pl.ds(off, CA)], dstb)
        pltpu.sync_copy(ew_hbm.at[pl.ds(off, CA)], ewb)

        def grp(g, _):
            o = pl.multiple_of(g * 16, 16)
            ewb[pl.ds(o, 16)] = jnp.maximum(ewb[pl.ds(o, 16)], zeros16f)
            return 0
        lax.fori_loop(0, CA // 16, grp, 0)
        pltpu.sync_copy(ewb, den_sh.at[dstb], add=True)
        return 0
    lax.fori_loop(0, ET // CA, chunk, 0)
    plsc.subcore_barrier()
    pltpu.sync_copy(den_sh.at[pl.ds(s * ROWS, ROWS)],
                    den_hbm.at[c, pl.ds(s * ROWS, ROWS)])


@functools.partial(
    pl.kernel,
    out_type=(jax.ShapeDtypeStruct((NC, NPAD, D), _F32),
              jax.ShapeDtypeStruct((NC, H, NPAD), _F32)),
    mesh=_MESH,
    compiler_params=pltpu.CompilerParams(needs_layout_passes=False),
    scratch_types=[
        pltpu.VMEM((CH,), _F32),
        pltpu.VMEM((CH,), _F32),
        pltpu.VMEM((H * 16,), _F32),
        pltpu.VMEM((CH,), _I32),
        pltpu.VMEM((CH,), _I32),
        pltpu.VMEM((CH,), _F32),
        pltpu.VMEM((CH, D), _F32),
        pltpu.VMEM((CH, D), _F32),
        pltpu.VMEM((CH, D), _F32),
        pltpu.VMEM((CH, D), _F32),
        pltpu.VMEM((H * CH,), _F32),
        pltpu.VMEM((ROWS,), _F32),
        pltpu.VMEM_SHARED((NPAD, D), _F32),
    ] + [pltpu.VMEM_SHARED((NPAD,), _F32) for _ in range(H)] + [
        pltpu.SemaphoreType.DMA,
        pltpu.SemaphoreType.DMA,
        pltpu.SemaphoreType.DMA,
    ],
)
def _phase_b(src_hbm, dst_hbm, ew_hbm, q_hbm, k_hbm, v_hbm, den0_hbm,
             den1_hbm, srep_hbm, t_hbm, dh_hbm,
             denb0, denb1, srepv, srcb, dstb, ewb, qb, kb, vb, mb, wTf, zb,
             t_sh, dh0, dh1, dh2, dh3, dh4, dh5, dh6, dh7,
             sem1, sem2, sem3):
    c = lax.axis_index("c")
    s = lax.axis_index("s")
    wid = s * NC + c
    dhs = (dh0, dh1, dh2, dh3, dh4, dh5, dh6, dh7)

    pltpu.sync_copy(srep_hbm, srepv)

    zeros16f = jnp.zeros((16,), _F32)

    def zmb(r, _):
        for dd in range(D // 16):
            mb[r, pl.ds(dd * 16, 16)] = zeros16f
        return 0
    lax.fori_loop(0, CH, zmb, 0)

    def zzb(i, _):
        zb[pl.ds(pl.multiple_of(i * 16, 16), 16)] = zeros16f
        return 0
    lax.fori_loop(0, ROWS // 16, zzb, 0)

    def zcp(j, _):
        pltpu.sync_copy(mb, t_sh.at[pl.ds(s * ROWS + j * CH, CH), :])
        return 0
    lax.fori_loop(0, ROWS // CH, zcp, 0)
    for h in range(H):
        pltpu.sync_copy(zb, dhs[h].at[pl.ds(s * ROWS, ROWS)])
    plsc.subcore_barrier()

    base = wid * ET
    quart = jnp.full((16,), 1.0 / np.sqrt(DK), _F32)
    epsv = jnp.full((16,), EPS, _F32)
    lanes0 = lax.iota(_I32, 16)

    def chunk(ci, _):
        off = base + ci * CH
        pltpu.sync_copy(src_hbm.at[pl.ds(off, CH)], srcb)
        pltpu.sync_copy(dst_hbm.at[pl.ds(off, CH)], dstb)
        pltpu.sync_copy(ew_hbm.at[pl.ds(off, CH)], ewb)
        cq = pltpu.async_copy(q_hbm.at[dstb], qb, sem1)
        ck = pltpu.async_copy(k_hbm.at[srcb], kb, sem2)
        cv = pltpu.async_copy(v_hbm.at[srcb], vb, sem3)
        cd0 = pltpu.async_copy(den0_hbm.at[dstb], denb0, sem1)
        cd1 = pltpu.async_copy(den1_hbm.at[dstb], denb1, sem2)
        cq.wait()
        ck.wait()
        cv.wait()
        cd0.wait()
        cd1.wait()

        def grp(g, _):
            o = pl.multiple_of(g * 16, 16)
            lanes = lanes0 + o
            e16 = jnp.maximum(ewb[pl.ds(o, 16)], jnp.zeros((16,), _F32))
            dsum = jnp.maximum(denb0[pl.ds(o, 16)] + denb1[pl.ds(o, 16)],
                               epsv)
            lw = _ln16(e16 / dsum + epsv)
            for h in range(H):
                acc = jnp.zeros((16,), _F32)
                for dd in range(DK):
                    col = jnp.full((16,), h * DK + dd, _I32)
                    qv = plsc.load_gather(qb, [lanes, col])
                    kv = plsc.load_gather(kb, [lanes, col])
                    acc = acc + qv * kv
                wv = jnp.exp((acc + lw) * srepv[pl.ds(h * 16, 16)] * quart)
                wTf[pl.ds(pl.multiple_of(h * CH + o, 16), 16)] = wv
                for dd in range(DK):
                    col = jnp.full((16,), h * DK + dd, _I32)
                    vv = plsc.load_gather(vb, [lanes, col])
                    plsc.store_scatter(mb, [lanes, col], vv * wv)
            return 0
        lax.fori_loop(0, CH // 16, grp, 0)
        pltpu.sync_copy(mb, t_sh.at[dstb], add=True)
        for h in range(H):
            pltpu.sync_copy(wTf.at[pl.ds(h * CH, CH)], dhs[h].at[dstb],
                            add=True)
        return 0
    lax.fori_loop(0, NCH, chunk, 0)
    plsc.subcore_barrier()
    pltpu.sync_copy(t_sh.at[pl.ds(s * ROWS, ROWS), :],
                    t_hbm.at[c, pl.ds(s * ROWS, ROWS), :])
    for h in range(H):
        pltpu.sync_copy(dhs[h].at[pl.ds(s * ROWS, ROWS)],
                        dh_hbm.at[c, h, pl.ds(s * ROWS, ROWS)])


def _out_body(tp_ref, dp_ref, x_ref, wa_ref, ba_ref, sk_ref, g_ref, b_ref,
              o_ref):
    t = tp_ref[0] + tp_ref[1]
    dh = dp_ref[0] + dp_ref[1]
    denr = jnp.concatenate(
        [jnp.broadcast_to(dh[:, h:h + 1], (_BN, DK)) for h in range(H)],
        axis=1)
    tn = t / jnp.maximum(denr, 1e-30)
    trans = jnp.dot(tn, wa_ref[...], preferred_element_type=_F32) + ba_ref[...]
    sk = sk_ref[0, 0]
    alpha = 1.0 / (1.0 + jnp.exp(-sk))
    o = trans * alpha + x_ref[...] * (1.0 - alpha)
    mu = jnp.mean(o, axis=-1, keepdims=True)
    oc = o - mu
    var = jnp.mean(oc * oc, axis=-1, keepdims=True)
    o_ref[...] = oc * lax.rsqrt(var + LN_EPS) * g_ref[...] + b_ref[...]


def _out(t_part, dh_part, x, Wa, ba, skip, ln_g, ln_b):
    return pl.pallas_call(
        _out_body,
        grid=(N // _BN,),
        in_specs=[
            pl.BlockSpec((NC, _BN, D), lambda i: (0, i, 0)),
            pl.BlockSpec((NC, _BN, H), lambda i: (0, i, 0)),
            pl.BlockSpec((_BN, D), lambda i: (i, 0)),
            pl.BlockSpec((D, D), lambda i: (0, 0)),
            pl.BlockSpec((1, D), lambda i: (0, 0)),
            pl.BlockSpec((1, 1), lambda i: (0, 0)),
            pl.BlockSpec((1, D), lambda i: (0, 0)),
            pl.BlockSpec((1, D), lambda i: (0, 0)),
        ],
        out_specs=pl.BlockSpec((_BN, D), lambda i: (i, 0)),
        out_shape=jax.ShapeDtypeStruct((N, D), _F32),
    )(t_part, dh_part, x, Wa, ba.reshape(1, D), skip.reshape(1, 1),
      ln_g.reshape(1, D), ln_b.reshape(1, D))


def kernel(x, edge_index, edge_weight, Wk, bk, Wq, bq, Wv, bv, Wa, ba,
           rel_att, rel_msg, rel_pri, skip, ln_g, ln_b):
    src = edge_index[0].astype(_I32)
    dst = edge_index[1].astype(_I32)
    ew = edge_weight.astype(_F32)
    srep = jnp.broadcast_to(rel_pri[:, None], (H, 16)).astype(_F32).reshape(H * 16)

    q, k2, v2 = _proj(x, Wq, bq, Wk, bk, Wv, bv, rel_att, rel_msg)
    den = _phase_a(dst, ew)
    t_part, dh_part = _phase_b(src, dst, ew, q, k2, v2, den[0], den[1], srep)
    dh_part = jnp.transpose(dh_part, (0, 2, 1))
    return _out(t_part, dh_part, x, Wa, ba, skip, ln_g, ln_b)


# speedup vs baseline: 13.9170x; 1.0868x over previous
import functools

import jax
import jax.numpy as jnp
import numpy as np
from jax import lax
from jax.experimental import pallas as pl
from jax.experimental.pallas import tpu as pltpu
from jax.experimental.pallas import tpu_sc as plsc

N = 10000
E = 320000
D = 128
H = 8
DK = D // H
EPS = 1e-9
LN_EPS = 1e-5

NC = 2
NS = 16
NW = NC * NS
NPAD = 10240
ROWS = NPAD // NS
ET = E // NW
CA = 400
CH = 80
NCH = ET // CH
NSLOT = 5

_I32 = jnp.int32
_F32 = jnp.float32


def _proj_body(x_ref, wq_ref, bq_ref, wk_ref, bk_ref, wv_ref, bv_ref,
               ra_ref, rm_ref, q_ref, k_ref, v_ref):
    x = x_ref[...]
    q_ref[...] = jnp.dot(x, wq_ref[...], preferred_element_type=_F32) + bq_ref[...]
    xk = jnp.dot(x, wk_ref[...], preferred_element_type=_F32) + bk_ref[...]
    xv = jnp.dot(x, wv_ref[...], preferred_element_type=_F32) + bv_ref[...]
    k_ref[...] = jnp.concatenate(
        [jnp.dot(xk[:, h * DK:(h + 1) * DK], ra_ref[h],
                 preferred_element_type=_F32) for h in range(H)], axis=1)
    v_ref[...] = jnp.concatenate(
        [jnp.dot(xv[:, h * DK:(h + 1) * DK], rm_ref[h],
                 preferred_element_type=_F32) for h in range(H)], axis=1)


_BN = 2000


def _proj(x, Wq, bq, Wk, bk, Wv, bv, rel_att, rel_msg):
    sds = jax.ShapeDtypeStruct((N, D), _F32)
    row_spec = pl.BlockSpec((_BN, D), lambda i: (i, 0))
    full = pl.BlockSpec((D, D), lambda i: (0, 0))
    bias = pl.BlockSpec((1, D), lambda i: (0, 0))
    rel = pl.BlockSpec((H, DK, DK), lambda i: (0, 0, 0))
    return pl.pallas_call(
        _proj_body,
        grid=(N // _BN,),
        in_specs=[row_spec, full, bias, full, bias, full, bias, rel, rel],
        out_specs=(row_spec, row_spec, row_spec),
        out_shape=(sds, sds, sds),
    )(x, Wq, bq.reshape(1, D), Wk, bk.reshape(1, D), Wv, bv.reshape(1, D),
      rel_att, rel_msg)


def _ln16(xv):
    b = plsc.bitcast(xv, _I32)
    ex = (b >> jnp.full((16,), 23, _I32)) - jnp.full((16,), 127, _I32)
    mb = (b & jnp.full((16,), 0x007FFFFF, _I32)) | jnp.full((16,), 0x3F800000, _I32)
    m = plsc.bitcast(mb, _F32)
    adj = m > jnp.full((16,), 1.4142135, _F32)
    m = jnp.where(adj, m * jnp.full((16,), 0.5, _F32), m)
    ex = jnp.where(adj, ex + jnp.full((16,), 1, _I32), ex)
    one = jnp.full((16,), 1.0, _F32)
    z = (m - one) / (m + one)
    z2 = z * z
    lnm = jnp.full((16,), 2.0, _F32) * z * (
        one + z2 * (jnp.full((16,), 1.0 / 3.0, _F32)
                    + z2 * (jnp.full((16,), 0.2, _F32)
                            + z2 * jnp.full((16,), 1.0 / 7.0, _F32))))
    return ex.astype(_F32) * jnp.full((16,), 0.6931471805599453, _F32) + lnm


_MESH = plsc.VectorSubcoreMesh(core_axis_name="c", subcore_axis_name="s")


@functools.partial(
    pl.kernel,
    out_type=jax.ShapeDtypeStruct((NC, NPAD), _F32),
    mesh=_MESH,
    compiler_params=pltpu.CompilerParams(needs_layout_passes=False),
    scratch_types=[
        pltpu.VMEM((CA,), _I32),
        pltpu.VMEM((CA,), _F32),
        pltpu.VMEM((ROWS,), _F32),
        pltpu.VMEM_SHARED((NPAD,), _F32),
    ],
)
def _phase_a(dst_hbm, ew_hbm, den_hbm, dstb, ewb, outb, den_sh):
    c = lax.axis_index("c")
    s = lax.axis_index("s")
    wid = s * NC + c
    zeros16f = jnp.zeros((16,), _F32)

    def zloop(i, _):
        outb[pl.ds(pl.multiple_of(i * 16, 16), 16)] = zeros16f
        return 0
    lax.fori_loop(0, ROWS // 16, zloop, 0)
    pltpu.sync_copy(outb, den_sh.at[pl.ds(s * ROWS, ROWS)])
    plsc.subcore_barrier()

    base = wid * ET

    def chunk(ci, _):
        off = base + ci * CA
        pltpu.sync_copy(dst_hbm.at[pl.ds(off, CA)], dstb)
        pltpu.sync_copy(ew_hbm.at[pl.ds(off, CA)], ewb)

        def grp(g, _):
            o = pl.multiple_of(g * 16, 16)
            ewb[pl.ds(o, 16)] = jnp.maximum(ewb[pl.ds(o, 16)], zeros16f)
            return 0
        lax.fori_loop(0, CA // 16, grp, 0)
        pltpu.sync_copy(ewb, den_sh.at[dstb], add=True)
        return 0
    lax.fori_loop(0, ET // CA, chunk, 0)
    plsc.subcore_barrier()
    pltpu.sync_copy(den_sh.at[pl.ds(s * ROWS, ROWS)],
                    den_hbm.at[c, pl.ds(s * ROWS, ROWS)])


@functools.partial(
    pl.kernel,
    out_type=(jax.ShapeDtypeStruct((NC, NPAD, D), _F32),
              jax.ShapeDtypeStruct((NC, H, NPAD), _F32)),
    mesh=_MESH,
    compiler_params=pltpu.CompilerParams(needs_layout_passes=False),
    scratch_types=[
        pltpu.VMEM((CH,), _F32),
        pltpu.VMEM((CH,), _F32),
        pltpu.VMEM((H * 16,), _F32),
    ] + [pltpu.VMEM((CH,), _I32) for _ in range(2 * NSLOT)]
      + [pltpu.VMEM((CH,), _F32) for _ in range(NSLOT)] + [
        pltpu.VMEM((CH, D), _F32),
        pltpu.VMEM((CH, D), _F32),
        pltpu.VMEM((CH, D), _F32),
        pltpu.VMEM((CH, D), _F32),
        pltpu.VMEM((H * CH,), _F32),
        pltpu.VMEM_SHARED((NPAD, D), _F32),
    ] + [pltpu.VMEM_SHARED((NPAD,), _F32) for _ in range(H)]
      + [pltpu.SemaphoreType.DMA for _ in range(NSLOT + 2)],
)
def _phase_b(src_hbm, dst_hbm, ew_hbm, q_hbm, k_hbm, v_hbm, den0_hbm,
             den1_hbm, srep_hbm, t_hbm, dh_hbm,
             denb0, denb1, srepv, *rest):
    srcbs = rest[0:NSLOT]
    dstbs = rest[NSLOT:2 * NSLOT]
    ewbs = rest[2 * NSLOT:3 * NSLOT]
    qb, kb, vb, mb, wTf, t_sh = rest[3 * NSLOT:3 * NSLOT + 6]
    dhs = rest[3 * NSLOT + 6:3 * NSLOT + 14]
    isems = rest[3 * NSLOT + 14:3 * NSLOT + 14 + NSLOT]
    gsem, ssem = rest[3 * NSLOT + 14 + NSLOT:]

    c = lax.axis_index("c")
    s = lax.axis_index("s")
    wid = s * NC + c

    pltpu.sync_copy(srep_hbm, srepv)

    zeros16f = jnp.zeros((16,), _F32)

    def zmb(r, _):
        for dd in range(D // 16):
            mb[r, pl.ds(dd * 16, 16)] = zeros16f
        return 0
    lax.fori_loop(0, CH, zmb, 0)

    def zwt(i, _):
        wTf[pl.ds(pl.multiple_of(i * 16, 16), 16)] = zeros16f
        return 0
    lax.fori_loop(0, H * CH // 16, zwt, 0)

    def zcp(j, _):
        pltpu.sync_copy(mb, t_sh.at[pl.ds(s * ROWS + j * CH, CH), :])
        return 0
    lax.fori_loop(0, ROWS // CH, zcp, 0)
    for h in range(H):
        pltpu.sync_copy(wTf, dhs[h].at[pl.ds(s * ROWS, ROWS)])
    plsc.subcore_barrier()

    base = wid * ET
    quart = jnp.full((16,), 1.0 / np.sqrt(DK), _F32)
    epsv = jnp.full((16,), EPS, _F32)
    lanes0 = lax.iota(_I32, 16)

    def issue_idx(ci, b):
        off = base + ci * CH
        pltpu.async_copy(src_hbm.at[pl.ds(off, CH)], srcbs[b], isems[b])
        pltpu.async_copy(dst_hbm.at[pl.ds(off, CH)], dstbs[b], isems[b])
        pltpu.async_copy(ew_hbm.at[pl.ds(off, CH)], ewbs[b], isems[b])

    def wait_idx(ci, b):
        off = base + ci * CH
        pltpu.make_async_copy(src_hbm.at[pl.ds(off, CH)], srcbs[b],
                              isems[b]).wait()
        pltpu.make_async_copy(dst_hbm.at[pl.ds(off, CH)], dstbs[b],
                              isems[b]).wait()
        pltpu.make_async_copy(ew_hbm.at[pl.ds(off, CH)], ewbs[b],
                              isems[b]).wait()

    def drain_scatters(bp):
        pltpu.make_async_copy(mb, t_sh.at[dstbs[bp]], ssem).wait()
        for h in range(H):
            pltpu.make_async_copy(wTf.at[pl.ds(h * CH, CH)],
                                  dhs[h].at[dstbs[bp]], ssem).wait()

    for b in range(NSLOT - 1):
        issue_idx(b, b)

    def super_chunk(sc, _):
        for b in range(NSLOT):
            ci = sc * NSLOT + b
            sb, db, eb = srcbs[b], dstbs[b], ewbs[b]
            wait_idx(ci, b)
            cq = pltpu.async_copy(q_hbm.at[db], qb, gsem)
            ck = pltpu.async_copy(k_hbm.at[sb], kb, gsem)
            cv = pltpu.async_copy(v_hbm.at[sb], vb, gsem)
            c0 = pltpu.async_copy(den0_hbm.at[db], denb0, gsem)
            c1 = pltpu.async_copy(den1_hbm.at[db], denb1, gsem)
            bp = (b - 1) % NSLOT
            if b == 0:
                @pl.when(sc > 0)
                def _():
                    drain_scatters(bp)
            else:
                drain_scatters(bp)

            @pl.when(ci + NSLOT - 1 < NCH)
            def _():
                issue_idx(ci + NSLOT - 1, bp)

            cq.wait()
            ck.wait()
            cv.wait()
            c0.wait()
            c1.wait()

            def grp(g, _):
                o = pl.multiple_of(g * 16, 16)
                lanes = lanes0 + o
                e16 = jnp.maximum(eb[pl.ds(o, 16)], jnp.zeros((16,), _F32))
                dsum = jnp.maximum(denb0[pl.ds(o, 16)] + denb1[pl.ds(o, 16)],
                                   epsv)
                lw = _ln16(e16 / dsum + epsv)
                for h in range(H):
                    acc = jnp.zeros((16,), _F32)
                    for dd in range(DK):
                        col = jnp.full((16,), h * DK + dd, _I32)
                        qv = plsc.load_gather(qb, [lanes, col])
                        kv = plsc.load_gather(kb, [lanes, col])
                        acc = acc + qv * kv
                    wv = jnp.exp((acc + lw) * srepv[pl.ds(h * 16, 16)] * quart)
                    wTf[pl.ds(pl.multiple_of(h * CH + o, 16), 16)] = wv
                    for dd in range(DK):
                        col = jnp.full((16,), h * DK + dd, _I32)
                        vv = plsc.load_gather(vb, [lanes, col])
                        plsc.store_scatter(mb, [lanes, col], vv * wv)
                return 0
            lax.fori_loop(0, CH // 16, grp, 0)
            pltpu.async_copy(mb, t_sh.at[db], ssem, add=True)
            for h in range(H):
                pltpu.async_copy(wTf.at[pl.ds(h * CH, CH)], dhs[h].at[db],
                                 ssem, add=True)
        return 0
    lax.fori_loop(0, NCH // NSLOT, super_chunk, 0)
    drain_scatters(NSLOT - 1)
    plsc.subcore_barrier()
    pltpu.sync_copy(t_sh.at[pl.ds(s * ROWS, ROWS), :],
                    t_hbm.at[c, pl.ds(s * ROWS, ROWS), :])
    for h in range(H):
        pltpu.sync_copy(dhs[h].at[pl.ds(s * ROWS, ROWS)],
                        dh_hbm.at[c, h, pl.ds(s * ROWS, ROWS)])


def _out_body(tp_ref, dp_ref, x_ref, wa_ref, ba_ref, sk_ref, g_ref, b_ref,
              o_ref):
    t = tp_ref[0] + tp_ref[1]
    dh = dp_ref[0] + dp_ref[1]
    denr = jnp.concatenate(
        [jnp.broadcast_to(dh[:, h:h + 1], (_BN, DK)) for h in range(H)],
        axis=1)
    tn = t / jnp.maximum(denr, 1e-30)
    trans = jnp.dot(tn, wa_ref[...], preferred_element_type=_F32) + ba_ref[...]
    sk = sk_ref[0, 0]
    alpha = 1.0 / (1.0 + jnp.exp(-sk))
    o = trans * alpha + x_ref[...] * (1.0 - alpha)
    mu = jnp.mean(o, axis=-1, keepdims=True)
    oc = o - mu
    var = jnp.mean(oc * oc, axis=-1, keepdims=True)
    o_ref[...] = oc * lax.rsqrt(var + LN_EPS) * g_ref[...] + b_ref[...]


def _out(t_part, dh_part, x, Wa, ba, skip, ln_g, ln_b):
    return pl.pallas_call(
        _out_body,
        grid=(N // _BN,),
        in_specs=[
            pl.BlockSpec((NC, _BN, D), lambda i: (0, i, 0)),
            pl.BlockSpec((NC, _BN, H), lambda i: (0, i, 0)),
            pl.BlockSpec((_BN, D), lambda i: (i, 0)),
            pl.BlockSpec((D, D), lambda i: (0, 0)),
            pl.BlockSpec((1, D), lambda i: (0, 0)),
            pl.BlockSpec((1, 1), lambda i: (0, 0)),
            pl.BlockSpec((1, D), lambda i: (0, 0)),
            pl.BlockSpec((1, D), lambda i: (0, 0)),
        ],
        out_specs=pl.BlockSpec((_BN, D), lambda i: (i, 0)),
        out_shape=jax.ShapeDtypeStruct((N, D), _F32),
    )(t_part, dh_part, x, Wa, ba.reshape(1, D), skip.reshape(1, 1),
      ln_g.reshape(1, D), ln_b.reshape(1, D))


def kernel(x, edge_index, edge_weight, Wk, bk, Wq, bq, Wv, bv, Wa, ba,
           rel_att, rel_msg, rel_pri, skip, ln_g, ln_b):
    src = edge_index[0].astype(_I32)
    dst = edge_index[1].astype(_I32)
    ew = edge_weight.astype(_F32)
    srep = jnp.broadcast_to(rel_pri[:, None], (H, 16)).astype(_F32).reshape(H * 16)

    q, k2, v2 = _proj(x, Wq, bq, Wk, bk, Wv, bv, rel_att, rel_msg)
    den = _phase_a(dst, ew)
    t_part, dh_part = _phase_b(src, dst, ew, q, k2, v2, den[0], den[1], srep)
    dh_part = jnp.transpose(dh_part, (0, 2, 1))
    return _out(t_part, dh_part, x, Wa, ba, skip, ln_g, ln_b)


# speedup vs baseline: 19.8266x; 1.4246x over previous
import functools

import jax
import jax.numpy as jnp
import numpy as np
from jax import lax
from jax.experimental import pallas as pl
from jax.experimental.pallas import tpu as pltpu
from jax.experimental.pallas import tpu_sc as plsc

N = 10000
E = 320000
D = 128
H = 8
DK = D // H
EPS = 1e-9
LN_EPS = 1e-5

NC = 2
NS = 16
NW = NC * NS
NPAD = 10240
ROWS = NPAD // NS
ET = E // NW
CA = 400
CH = 80
NCH = ET // CH
NSLOT = 5

_I32 = jnp.int32
_F32 = jnp.float32


def _proj_body(x_ref, wq_ref, bq_ref, wk_ref, bk_ref, wv_ref, bv_ref,
               ra_ref, rm_ref, q_ref, k_ref, v_ref):
    x = x_ref[...]
    q_ref[...] = jnp.dot(x, wq_ref[...], preferred_element_type=_F32) + bq_ref[...]
    xk = jnp.dot(x, wk_ref[...], preferred_element_type=_F32) + bk_ref[...]
    xv = jnp.dot(x, wv_ref[...], preferred_element_type=_F32) + bv_ref[...]
    k_ref[...] = jnp.concatenate(
        [jnp.dot(xk[:, h * DK:(h + 1) * DK], ra_ref[h],
                 preferred_element_type=_F32) for h in range(H)], axis=1)
    v_ref[...] = jnp.concatenate(
        [jnp.dot(xv[:, h * DK:(h + 1) * DK], rm_ref[h],
                 preferred_element_type=_F32) for h in range(H)], axis=1)


_BN = 2000


def _proj(x, Wq, bq, Wk, bk, Wv, bv, rel_att, rel_msg):
    sds = jax.ShapeDtypeStruct((N, D), _F32)
    row_spec = pl.BlockSpec((_BN, D), lambda i: (i, 0))
    full = pl.BlockSpec((D, D), lambda i: (0, 0))
    bias = pl.BlockSpec((1, D), lambda i: (0, 0))
    rel = pl.BlockSpec((H, DK, DK), lambda i: (0, 0, 0))
    return pl.pallas_call(
        _proj_body,
        grid=(N // _BN,),
        in_specs=[row_spec, full, bias, full, bias, full, bias, rel, rel],
        out_specs=(row_spec, row_spec, row_spec),
        out_shape=(sds, sds, sds),
    )(x, Wq, bq.reshape(1, D), Wk, bk.reshape(1, D), Wv, bv.reshape(1, D),
      rel_att, rel_msg)


_DNUMS = lax.GatherDimensionNumbers(
    offset_dims=(), collapsed_slice_dims=(0,), start_index_map=(0,))


def _splat(v, lane):
    idx = jnp.full((16, 1), lane, _I32) if isinstance(lane, int) else \
        jnp.broadcast_to(lane.astype(_I32), (16,)).reshape(16, 1)
    return lax.gather(v, idx, _DNUMS, (1,),
                      mode=lax.GatherScatterMode.PROMISE_IN_BOUNDS)


def _ln16(xv):
    b = plsc.bitcast(xv, _I32)
    ex = (b >> jnp.full((16,), 23, _I32)) - jnp.full((16,), 127, _I32)
    mb = (b & jnp.full((16,), 0x007FFFFF, _I32)) | jnp.full((16,), 0x3F800000, _I32)
    m = plsc.bitcast(mb, _F32)
    adj = m > jnp.full((16,), 1.4142135, _F32)
    m = jnp.where(adj, m * jnp.full((16,), 0.5, _F32), m)
    ex = jnp.where(adj, ex + jnp.full((16,), 1, _I32), ex)
    one = jnp.full((16,), 1.0, _F32)
    z = (m - one) / (m + one)
    z2 = z * z
    lnm = jnp.full((16,), 2.0, _F32) * z * (
        one + z2 * (jnp.full((16,), 1.0 / 3.0, _F32)
                    + z2 * (jnp.full((16,), 0.2, _F32)
                            + z2 * jnp.full((16,), 1.0 / 7.0, _F32))))
    return ex.astype(_F32) * jnp.full((16,), 0.6931471805599453, _F32) + lnm


_MESH = plsc.VectorSubcoreMesh(core_axis_name="c", subcore_axis_name="s")


@functools.partial(
    pl.kernel,
    out_type=jax.ShapeDtypeStruct((NC, NPAD), _F32),
    mesh=_MESH,
    compiler_params=pltpu.CompilerParams(needs_layout_passes=False),
    scratch_types=[
        pltpu.VMEM((CA,), _I32),
        pltpu.VMEM((CA,), _F32),
        pltpu.VMEM((ROWS,), _F32),
        pltpu.VMEM_SHARED((NPAD,), _F32),
    ],
)
def _phase_a(dst_hbm, ew_hbm, den_hbm, dstb, ewb, outb, den_sh):
    c = lax.axis_index("c")
    s = lax.axis_index("s")
    wid = s * NC + c
    zeros16f = jnp.zeros((16,), _F32)

    def zloop(i, _):
        outb[pl.ds(pl.multiple_of(i * 16, 16), 16)] = zeros16f
        return 0
    lax.fori_loop(0, ROWS // 16, zloop, 0)
    pltpu.sync_copy(outb, den_sh.at[pl.ds(s * ROWS, ROWS)])
    plsc.subcore_barrier()

    base = wid * ET

    def chunk(ci, _):
        off = base + ci * CA
        pltpu.sync_copy(dst_hbm.at[pl.ds(off, CA)], dstb)
        pltpu.sync_copy(ew_hbm.at[pl.ds(off, CA)], ewb)

        def grp(g, _):
            o = pl.multiple_of(g * 16, 16)
            ewb[pl.ds(o, 16)] = jnp.maximum(ewb[pl.ds(o, 16)], zeros16f)
            return 0
        lax.fori_loop(0, CA // 16, grp, 0)
        pltpu.sync_copy(ewb, den_sh.at[dstb], add=True)
        return 0
    lax.fori_loop(0, ET // CA, chunk, 0)
    plsc.subcore_barrier()
    pltpu.sync_copy(den_sh.at[pl.ds(s * ROWS, ROWS)],
                    den_hbm.at[c, pl.ds(s * ROWS, ROWS)])


@functools.partial(
    pl.kernel,
    out_type=(jax.ShapeDtypeStruct((NC, NPAD, D), _F32),
              jax.ShapeDtypeStruct((NC, H, NPAD), _F32)),
    mesh=_MESH,
    compiler_params=pltpu.CompilerParams(needs_layout_passes=False),
    scratch_types=[
        pltpu.VMEM((CH,), _F32),
        pltpu.VMEM((CH,), _F32),
        pltpu.VMEM((H * 16,), _F32),
    ] + [pltpu.VMEM((CH,), _I32) for _ in range(2 * NSLOT)]
      + [pltpu.VMEM((CH,), _F32) for _ in range(NSLOT)] + [
        pltpu.VMEM((CH, D), _F32),
        pltpu.VMEM((CH, D), _F32),
        pltpu.VMEM((CH, D), _F32),
        pltpu.VMEM((CH, D), _F32),
        pltpu.VMEM((H * CH,), _F32),
        pltpu.VMEM_SHARED((NPAD, D), _F32),
    ] + [pltpu.VMEM_SHARED((NPAD,), _F32) for _ in range(H)]
      + [pltpu.SemaphoreType.DMA for _ in range(NSLOT + 2)],
)
def _phase_b(src_hbm, dst_hbm, ew_hbm, q_hbm, k_hbm, v_hbm, den0_hbm,
             den1_hbm, srep_hbm, t_hbm, dh_hbm,
             denb0, denb1, srepv, *rest):
    srcbs = rest[0:NSLOT]
    dstbs = rest[NSLOT:2 * NSLOT]
    ewbs = rest[2 * NSLOT:3 * NSLOT]
    qb, kb, vb, mb, wTf, t_sh = rest[3 * NSLOT:3 * NSLOT + 6]
    dhs = rest[3 * NSLOT + 6:3 * NSLOT + 14]
    isems = rest[3 * NSLOT + 14:3 * NSLOT + 14 + NSLOT]
    gsem, ssem = rest[3 * NSLOT + 14 + NSLOT:]

    c = lax.axis_index("c")
    s = lax.axis_index("s")
    wid = s * NC + c

    pltpu.sync_copy(srep_hbm, srepv)

    zeros16f = jnp.zeros((16,), _F32)

    def zmb(r, _):
        for dd in range(D // 16):
            mb[r, pl.ds(dd * 16, 16)] = zeros16f
        return 0
    lax.fori_loop(0, CH, zmb, 0)

    def zwt(i, _):
        wTf[pl.ds(pl.multiple_of(i * 16, 16), 16)] = zeros16f
        return 0
    lax.fori_loop(0, H * CH // 16, zwt, 0)

    def zcp(j, _):
        pltpu.sync_copy(mb, t_sh.at[pl.ds(s * ROWS + j * CH, CH), :])
        return 0
    lax.fori_loop(0, ROWS // CH, zcp, 0)
    for h in range(H):
        pltpu.sync_copy(wTf, dhs[h].at[pl.ds(s * ROWS, ROWS)])
    plsc.subcore_barrier()

    base = wid * ET
    quart = jnp.full((16,), 1.0 / np.sqrt(DK), _F32)
    epsv = jnp.full((16,), EPS, _F32)
    lanes0 = lax.iota(_I32, 16)
    mask0 = lanes0 == jnp.zeros((16,), _I32)
    svs = [srepv[pl.ds(h * 16, 16)] * quart for h in range(H)]

    def issue_idx(ci, b):
        off = base + ci * CH
        pltpu.async_copy(src_hbm.at[pl.ds(off, CH)], srcbs[b], isems[b])
        pltpu.async_copy(dst_hbm.at[pl.ds(off, CH)], dstbs[b], isems[b])
        pltpu.async_copy(ew_hbm.at[pl.ds(off, CH)], ewbs[b], isems[b])

    def wait_idx(ci, b):
        off = base + ci * CH
        pltpu.make_async_copy(src_hbm.at[pl.ds(off, CH)], srcbs[b],
                              isems[b]).wait()
        pltpu.make_async_copy(dst_hbm.at[pl.ds(off, CH)], dstbs[b],
                              isems[b]).wait()
        pltpu.make_async_copy(ew_hbm.at[pl.ds(off, CH)], ewbs[b],
                              isems[b]).wait()

    def drain_scatters(bp):
        pltpu.make_async_copy(mb, t_sh.at[dstbs[bp]], ssem).wait()
        for h in range(H):
            pltpu.make_async_copy(wTf.at[pl.ds(h * CH, CH)],
                                  dhs[h].at[dstbs[bp]], ssem).wait()

    for b in range(NSLOT - 1):
        issue_idx(b, b)

    def super_chunk(sc, _):
        for b in range(NSLOT):
            ci = sc * NSLOT + b
            sb, db, eb = srcbs[b], dstbs[b], ewbs[b]
            wait_idx(ci, b)
            cq = pltpu.async_copy(q_hbm.at[db], qb, gsem)
            ck = pltpu.async_copy(k_hbm.at[sb], kb, gsem)
            cv = pltpu.async_copy(v_hbm.at[sb], vb, gsem)
            c0 = pltpu.async_copy(den0_hbm.at[db], denb0, gsem)
            c1 = pltpu.async_copy(den1_hbm.at[db], denb1, gsem)
            bp = (b - 1) % NSLOT
            if b == 0:
                @pl.when(sc > 0)
                def _():
                    drain_scatters(bp)
            else:
                drain_scatters(bp)

            @pl.when(ci + NSLOT - 1 < NCH)
            def _():
                issue_idx(ci + NSLOT - 1, bp)

            cq.wait()
            ck.wait()
            cv.wait()
            c0.wait()
            c1.wait()

            @pl.loop(0, CH // 16, unroll=1)
            def grp(g):
                o = pl.multiple_of(g * 16, 16)
                e16 = jnp.maximum(eb[pl.ds(o, 16)], jnp.zeros((16,), _F32))
                dsum = jnp.maximum(denb0[pl.ds(o, 16)] + denb1[pl.ds(o, 16)],
                                   epsv)
                lw = _ln16(e16 / dsum + epsv)

                def edge(e, _):
                    r = o + e
                    lwspl = _splat(lw, e)
                    for h in range(H):
                        qv = qb[r, pl.ds(h * DK, DK)]
                        kv = kb[r, pl.ds(h * DK, DK)]
                        sc = plsc.cumsum(qv * kv)
                        wv = jnp.exp((_splat(sc, 15) + lwspl) * svs[h])
                        vv = vb[r, pl.ds(h * DK, DK)]
                        mb[r, pl.ds(h * DK, DK)] = vv * wv
                        plsc.store_scatter(
                            wTf, [jnp.full((16,), h * CH, _I32) + r], wv,
                            mask=mask0)
                    return 0
                lax.fori_loop(0, 16, edge, 0)
            pltpu.async_copy(mb, t_sh.at[db], ssem, add=True)
            for h in range(H):
                pltpu.async_copy(wTf.at[pl.ds(h * CH, CH)], dhs[h].at[db],
                                 ssem, add=True)
        return 0
    lax.fori_loop(0, NCH // NSLOT, super_chunk, 0)
    drain_scatters(NSLOT - 1)
    plsc.subcore_barrier()
    pltpu.sync_copy(t_sh.at[pl.ds(s * ROWS, ROWS), :],
                    t_hbm.at[c, pl.ds(s * ROWS, ROWS), :])
    for h in range(H):
        pltpu.sync_copy(dhs[h].at[pl.ds(s * ROWS, ROWS)],
                        dh_hbm.at[c, h, pl.ds(s * ROWS, ROWS)])


def _out_body(tp_ref, dp_ref, x_ref, wa_ref, ba_ref, sk_ref, g_ref, b_ref,
              o_ref):
    t = tp_ref[0] + tp_ref[1]
    dh = dp_ref[0] + dp_ref[1]
    denr = jnp.concatenate(
        [jnp.broadcast_to(dh[:, h:h + 1], (_BN, DK)) for h in range(H)],
        axis=1)
    tn = t / jnp.maximum(denr, 1e-30)
    trans = jnp.dot(tn, wa_ref[...], preferred_element_type=_F32) + ba_ref[...]
    sk = sk_ref[0, 0]
    alpha = 1.0 / (1.0 + jnp.exp(-sk))
    o = trans * alpha + x_ref[...] * (1.0 - alpha)
    mu = jnp.mean(o, axis=-1, keepdims=True)
    oc = o - mu
    var = jnp.mean(oc * oc, axis=-1, keepdims=True)
    o_ref[...] = oc * lax.rsqrt(var + LN_EPS) * g_ref[...] + b_ref[...]


def _out(t_part, dh_part, x, Wa, ba, skip, ln_g, ln_b):
    return pl.pallas_call(
        _out_body,
        grid=(N // _BN,),
        in_specs=[
            pl.BlockSpec((NC, _BN, D), lambda i: (0, i, 0)),
            pl.BlockSpec((NC, _BN, H), lambda i: (0, i, 0)),
            pl.BlockSpec((_BN, D), lambda i: (i, 0)),
            pl.BlockSpec((D, D), lambda i: (0, 0)),
            pl.BlockSpec((1, D), lambda i: (0, 0)),
            pl.BlockSpec((1, 1), lambda i: (0, 0)),
            pl.BlockSpec((1, D), lambda i: (0, 0)),
            pl.BlockSpec((1, D), lambda i: (0, 0)),
        ],
        out_specs=pl.BlockSpec((_BN, D), lambda i: (i, 0)),
        out_shape=jax.ShapeDtypeStruct((N, D), _F32),
    )(t_part, dh_part, x, Wa, ba.reshape(1, D), skip.reshape(1, 1),
      ln_g.reshape(1, D), ln_b.reshape(1, D))


def kernel(x, edge_index, edge_weight, Wk, bk, Wq, bq, Wv, bv, Wa, ba,
           rel_att, rel_msg, rel_pri, skip, ln_g, ln_b):
    src = edge_index[0].astype(_I32)
    dst = edge_index[1].astype(_I32)
    ew = edge_weight.astype(_F32)
    srep = jnp.broadcast_to(rel_pri[:, None], (H, 16)).astype(_F32).reshape(H * 16)

    q, k2, v2 = _proj(x, Wq, bq, Wk, bk, Wv, bv, rel_att, rel_msg)
    den = _phase_a(dst, ew)
    t_part, dh_part = _phase_b(src, dst, ew, q, k2, v2, den[0], den[1], srep)
    dh_part = jnp.transpose(dh_part, (0, 2, 1))
    return _out(t_part, dh_part, x, Wa, ba, skip, ln_g, ln_b)


# speedup vs baseline: 61.9324x; 3.1237x over previous
import functools

import jax
import jax.numpy as jnp
import numpy as np
from jax import lax
from jax.experimental import pallas as pl
from jax.experimental.pallas import tpu as pltpu
from jax.experimental.pallas import tpu_sc as plsc

N = 10000
E = 320000
D = 128
H = 8
DK = D // H
EPS = 1e-9
LN_EPS = 1e-5

NC = 2
NS = 16
NW = NC * NS
NPAD = 10240
ROWS = NPAD // NS
ET = E // NW
CA = 400
CH = 80
NCH = ET // CH
NSLOT = 5

_I32 = jnp.int32
_F32 = jnp.float32


def _proj_body(x_ref, wq_ref, bq_ref, wk_ref, bk_ref, wv_ref, bv_ref,
               ra_ref, rm_ref, q_ref, k_ref, v_ref):
    x = x_ref[...]
    q_ref[...] = jnp.dot(x, wq_ref[...], preferred_element_type=_F32) + bq_ref[...]
    xk = jnp.dot(x, wk_ref[...], preferred_element_type=_F32) + bk_ref[...]
    xv = jnp.dot(x, wv_ref[...], preferred_element_type=_F32) + bv_ref[...]
    k_ref[...] = jnp.concatenate(
        [jnp.dot(xk[:, h * DK:(h + 1) * DK], ra_ref[h],
                 preferred_element_type=_F32) for h in range(H)], axis=1)
    v_ref[...] = jnp.concatenate(
        [jnp.dot(xv[:, h * DK:(h + 1) * DK], rm_ref[h],
                 preferred_element_type=_F32) for h in range(H)], axis=1)


_BN = 2000


def _proj(x, Wq, bq, Wk, bk, Wv, bv, rel_att, rel_msg):
    sds = jax.ShapeDtypeStruct((N, D), _F32)
    row_spec = pl.BlockSpec((_BN, D), lambda i: (i, 0))
    full = pl.BlockSpec((D, D), lambda i: (0, 0))
    bias = pl.BlockSpec((1, D), lambda i: (0, 0))
    rel = pl.BlockSpec((H, DK, DK), lambda i: (0, 0, 0))
    return pl.pallas_call(
        _proj_body,
        grid=(N // _BN,),
        in_specs=[row_spec, full, bias, full, bias, full, bias, rel, rel],
        out_specs=(row_spec, row_spec, row_spec),
        out_shape=(sds, sds, sds),
    )(x, Wq, bq.reshape(1, D), Wk, bk.reshape(1, D), Wv, bv.reshape(1, D),
      rel_att, rel_msg)


_DNUMS = lax.GatherDimensionNumbers(
    offset_dims=(), collapsed_slice_dims=(0,), start_index_map=(0,))


def _splat(v, lane):
    idx = jnp.full((16, 1), lane, _I32) if isinstance(lane, int) else \
        jnp.broadcast_to(lane.astype(_I32), (16,)).reshape(16, 1)
    return lax.gather(v, idx, _DNUMS, (1,),
                      mode=lax.GatherScatterMode.PROMISE_IN_BOUNDS)


def _ln16(xv):
    b = plsc.bitcast(xv, _I32)
    ex = (b >> jnp.full((16,), 23, _I32)) - jnp.full((16,), 127, _I32)
    mb = (b & jnp.full((16,), 0x007FFFFF, _I32)) | jnp.full((16,), 0x3F800000, _I32)
    m = plsc.bitcast(mb, _F32)
    adj = m > jnp.full((16,), 1.4142135, _F32)
    m = jnp.where(adj, m * jnp.full((16,), 0.5, _F32), m)
    ex = jnp.where(adj, ex + jnp.full((16,), 1, _I32), ex)
    one = jnp.full((16,), 1.0, _F32)
    z = (m - one) / (m + one)
    z2 = z * z
    lnm = jnp.full((16,), 2.0, _F32) * z * (
        one + z2 * (jnp.full((16,), 1.0 / 3.0, _F32)
                    + z2 * (jnp.full((16,), 0.2, _F32)
                            + z2 * jnp.full((16,), 1.0 / 7.0, _F32))))
    return ex.astype(_F32) * jnp.full((16,), 0.6931471805599453, _F32) + lnm


_MESH = plsc.VectorSubcoreMesh(core_axis_name="c", subcore_axis_name="s")


@functools.partial(
    pl.kernel,
    out_type=jax.ShapeDtypeStruct((NC, NPAD), _F32),
    mesh=_MESH,
    compiler_params=pltpu.CompilerParams(needs_layout_passes=False),
    scratch_types=[
        pltpu.VMEM((CA,), _I32),
        pltpu.VMEM((CA,), _F32),
        pltpu.VMEM((ROWS,), _F32),
        pltpu.VMEM_SHARED((NPAD,), _F32),
    ],
)
def _phase_a(dst_hbm, ew_hbm, den_hbm, dstb, ewb, outb, den_sh):
    c = lax.axis_index("c")
    s = lax.axis_index("s")
    wid = s * NC + c
    zeros16f = jnp.zeros((16,), _F32)

    def zloop(i, _):
        outb[pl.ds(pl.multiple_of(i * 16, 16), 16)] = zeros16f
        return 0
    lax.fori_loop(0, ROWS // 16, zloop, 0)
    pltpu.sync_copy(outb, den_sh.at[pl.ds(s * ROWS, ROWS)])
    plsc.subcore_barrier()

    base = wid * ET

    def chunk(ci, _):
        off = base + ci * CA
        pltpu.sync_copy(dst_hbm.at[pl.ds(off, CA)], dstb)
        pltpu.sync_copy(ew_hbm.at[pl.ds(off, CA)], ewb)

        def grp(g, _):
            o = pl.multiple_of(g * 16, 16)
            ewb[pl.ds(o, 16)] = jnp.maximum(ewb[pl.ds(o, 16)], zeros16f)
            return 0
        lax.fori_loop(0, CA // 16, grp, 0)
        pltpu.sync_copy(ewb, den_sh.at[dstb], add=True)
        return 0
    lax.fori_loop(0, ET // CA, chunk, 0)
    plsc.subcore_barrier()
    pltpu.sync_copy(den_sh.at[pl.ds(s * ROWS, ROWS)],
                    den_hbm.at[c, pl.ds(s * ROWS, ROWS)])


@functools.partial(
    pl.kernel,
    out_type=(jax.ShapeDtypeStruct((NC, NPAD, D), _F32),
              jax.ShapeDtypeStruct((NC, H, NPAD), _F32)),
    mesh=_MESH,
    compiler_params=pltpu.CompilerParams(needs_layout_passes=False),
    scratch_types=[
        pltpu.VMEM((CH,), _F32),
        pltpu.VMEM((CH,), _F32),
        pltpu.VMEM((H * 16,), _F32),
    ] + [pltpu.VMEM((CH,), _I32) for _ in range(2 * NSLOT)]
      + [pltpu.VMEM((CH,), _F32) for _ in range(NSLOT)] + [
        pltpu.VMEM((CH, D), _F32),
        pltpu.VMEM((CH, D), _F32),
        pltpu.VMEM((CH, D), _F32),
        pltpu.VMEM((CH, D), _F32),
        pltpu.VMEM((H * CH,), _F32),
        pltpu.VMEM_SHARED((NPAD, D), _F32),
    ] + [pltpu.VMEM_SHARED((NPAD,), _F32) for _ in range(H)]
      + [pltpu.SemaphoreType.DMA for _ in range(NSLOT + 2)],
)
def _phase_b(src_hbm, dst_hbm, ew_hbm, q_hbm, k_hbm, v_hbm, den0_hbm,
             den1_hbm, srep_hbm, t_hbm, dh_hbm,
             denb0, denb1, srepv, *rest):
    srcbs = rest[0:NSLOT]
    dstbs = rest[NSLOT:2 * NSLOT]
    ewbs = rest[2 * NSLOT:3 * NSLOT]
    qb, kb, vb, mb, wTf, t_sh = rest[3 * NSLOT:3 * NSLOT + 6]
    dhs = rest[3 * NSLOT + 6:3 * NSLOT + 14]
    isems = rest[3 * NSLOT + 14:3 * NSLOT + 14 + NSLOT]
    gsem, ssem = rest[3 * NSLOT + 14 + NSLOT:]

    c = lax.axis_index("c")
    s = lax.axis_index("s")
    wid = s * NC + c

    pltpu.sync_copy(srep_hbm, srepv)

    zeros16f = jnp.zeros((16,), _F32)

    def zmb(r, _):
        for dd in range(D // 16):
            mb[r, pl.ds(dd * 16, 16)] = zeros16f
        return 0
    lax.fori_loop(0, CH, zmb, 0)

    def zwt(i, _):
        wTf[pl.ds(pl.multiple_of(i * 16, 16), 16)] = zeros16f
        return 0
    lax.fori_loop(0, H * CH // 16, zwt, 0)

    def zcp(j, _):
        pltpu.sync_copy(mb, t_sh.at[pl.ds(s * ROWS + j * CH, CH), :])
        return 0
    lax.fori_loop(0, ROWS // CH, zcp, 0)
    for h in range(H):
        pltpu.sync_copy(wTf, dhs[h].at[pl.ds(s * ROWS, ROWS)])
    plsc.subcore_barrier()

    base = wid * ET
    quart = jnp.full((16,), 1.0 / np.sqrt(DK), _F32)
    epsv = jnp.full((16,), EPS, _F32)
    lanes0 = lax.iota(_I32, 16)
    mask0 = lanes0 == jnp.zeros((16,), _I32)
    svs = [srepv[pl.ds(h * 16, 16)] * quart for h in range(H)]

    def issue_idx(ci, b):
        off = base + ci * CH
        pltpu.async_copy(src_hbm.at[pl.ds(off, CH)], srcbs[b], isems[b])
        pltpu.async_copy(dst_hbm.at[pl.ds(off, CH)], dstbs[b], isems[b])
        pltpu.async_copy(ew_hbm.at[pl.ds(off, CH)], ewbs[b], isems[b])

    def wait_idx(ci, b):
        off = base + ci * CH
        pltpu.make_async_copy(src_hbm.at[pl.ds(off, CH)], srcbs[b],
                              isems[b]).wait()
        pltpu.make_async_copy(dst_hbm.at[pl.ds(off, CH)], dstbs[b],
                              isems[b]).wait()
        pltpu.make_async_copy(ew_hbm.at[pl.ds(off, CH)], ewbs[b],
                              isems[b]).wait()

    def drain_scatters(bp):
        pltpu.make_async_copy(mb, t_sh.at[dstbs[bp]], ssem).wait()
        for h in range(H):
            pltpu.make_async_copy(wTf.at[pl.ds(h * CH, CH)],
                                  dhs[h].at[dstbs[bp]], ssem).wait()

    for b in range(NSLOT - 1):
        issue_idx(b, b)

    def super_chunk(sc, _):
        for b in range(NSLOT):
            ci = sc * NSLOT + b
            sb, db, eb = srcbs[b], dstbs[b], ewbs[b]
            wait_idx(ci, b)
            cq = pltpu.async_copy(q_hbm.at[db], qb, gsem)
            ck = pltpu.async_copy(k_hbm.at[sb], kb, gsem)
            cv = pltpu.async_copy(v_hbm.at[sb], vb, gsem)
            c0 = pltpu.async_copy(den0_hbm.at[db], denb0, gsem)
            c1 = pltpu.async_copy(den1_hbm.at[db], denb1, gsem)
            bp = (b - 1) % NSLOT
            if b == 0:
                @pl.when(sc > 0)
                def _():
                    drain_scatters(bp)
            else:
                drain_scatters(bp)

            @pl.when(ci + NSLOT - 1 < NCH)
            def _():
                issue_idx(ci + NSLOT - 1, bp)

            cq.wait()
            ck.wait()
            cv.wait()
            c0.wait()
            c1.wait()

            @pl.loop(0, CH // 16, unroll=1)
            def grp(g):
                o = pl.multiple_of(g * 16, 16)
                e16 = jnp.maximum(eb[pl.ds(o, 16)], jnp.zeros((16,), _F32))
                dsum = jnp.maximum(denb0[pl.ds(o, 16)] + denb1[pl.ds(o, 16)],
                                   epsv)
                lw = _ln16(e16 / dsum + epsv)

                @plsc.parallel_loop(0, 16, 1, unroll=4)
                def edge(e):
                    r = o + e
                    lwspl = _splat(lw, e)
                    for h in range(H):
                        qv = qb[r, pl.ds(h * DK, DK)]
                        kv = kb[r, pl.ds(h * DK, DK)]
                        sc = plsc.cumsum(qv * kv)
                        wv = jnp.exp((_splat(sc, 15) + lwspl) * svs[h])
                        vv = vb[r, pl.ds(h * DK, DK)]
                        mb[r, pl.ds(h * DK, DK)] = vv * wv
                        plsc.store_scatter(
                            wTf, [jnp.full((16,), h * CH, _I32) + r], wv,
                            mask=mask0)
            pltpu.async_copy(mb, t_sh.at[db], ssem, add=True)
            for h in range(H):
                pltpu.async_copy(wTf.at[pl.ds(h * CH, CH)], dhs[h].at[db],
                                 ssem, add=True)
        return 0
    lax.fori_loop(0, NCH // NSLOT, super_chunk, 0)
    drain_scatters(NSLOT - 1)
    plsc.subcore_barrier()
    pltpu.sync_copy(t_sh.at[pl.ds(s * ROWS, ROWS), :],
                    t_hbm.at[c, pl.ds(s * ROWS, ROWS), :])
    for h in range(H):
        pltpu.sync_copy(dhs[h].at[pl.ds(s * ROWS, ROWS)],
                        dh_hbm.at[c, h, pl.ds(s * ROWS, ROWS)])


def _out_body(tp_ref, dp_ref, x_ref, wa_ref, ba_ref, sk_ref, g_ref, b_ref,
              o_ref):
    t = tp_ref[0] + tp_ref[1]
    dh = dp_ref[0] + dp_ref[1]
    denr = jnp.concatenate(
        [jnp.broadcast_to(dh[:, h:h + 1], (_BN, DK)) for h in range(H)],
        axis=1)
    tn = t / jnp.maximum(denr, 1e-30)
    trans = jnp.dot(tn, wa_ref[...], preferred_element_type=_F32) + ba_ref[...]
    sk = sk_ref[0, 0]
    alpha = 1.0 / (1.0 + jnp.exp(-sk))
    o = trans * alpha + x_ref[...] * (1.0 - alpha)
    mu = jnp.mean(o, axis=-1, keepdims=True)
    oc = o - mu
    var = jnp.mean(oc * oc, axis=-1, keepdims=True)
    o_ref[...] = oc * lax.rsqrt(var + LN_EPS) * g_ref[...] + b_ref[...]


def _out(t_part, dh_part, x, Wa, ba, skip, ln_g, ln_b):
    return pl.pallas_call(
        _out_body,
        grid=(N // _BN,),
        in_specs=[
            pl.BlockSpec((NC, _BN, D), lambda i: (0, i, 0)),
            pl.BlockSpec((NC, _BN, H), lambda i: (0, i, 0)),
            pl.BlockSpec((_BN, D), lambda i: (i, 0)),
            pl.BlockSpec((D, D), lambda i: (0, 0)),
            pl.BlockSpec((1, D), lambda i: (0, 0)),
            pl.BlockSpec((1, 1), lambda i: (0, 0)),
            pl.BlockSpec((1, D), lambda i: (0, 0)),
            pl.BlockSpec((1, D), lambda i: (0, 0)),
        ],
        out_specs=pl.BlockSpec((_BN, D), lambda i: (i, 0)),
        out_shape=jax.ShapeDtypeStruct((N, D), _F32),
    )(t_part, dh_part, x, Wa, ba.reshape(1, D), skip.reshape(1, 1),
      ln_g.reshape(1, D), ln_b.reshape(1, D))


def kernel(x, edge_index, edge_weight, Wk, bk, Wq, bq, Wv, bv, Wa, ba,
           rel_att, rel_msg, rel_pri, skip, ln_g, ln_b):
    src = edge_index[0].astype(_I32)
    dst = edge_index[1].astype(_I32)
    ew = edge_weight.astype(_F32)
    srep = jnp.broadcast_to(rel_pri[:, None], (H, 16)).astype(_F32).reshape(H * 16)

    q, k2, v2 = _proj(x, Wq, bq, Wk, bk, Wv, bv, rel_att, rel_msg)
    den = _phase_a(dst, ew)
    t_part, dh_part = _phase_b(src, dst, ew, q, k2, v2, den[0], den[1], srep)
    dh_part = jnp.transpose(dh_part, (0, 2, 1))
    return _out(t_part, dh_part, x, Wa, ba, skip, ln_g, ln_b)
